# baseline (speedup 1.0000x reference)
import functools
import jax
import jax.numpy as jnp
import numpy as np
from jax import lax
from jax.experimental import pallas as pl
from jax.experimental.pallas import tpu as pltpu
from jax.experimental.pallas import tpu_sc as plsc

DIM = 128
NUM_ENT = 10000
NUM_REL = 200
N_TRIPLES = 160000
BSZ = 16
MAX_ACTIVE = 400
THRES = 0.7
L = 16
ENT_CHUNKS = NUM_ENT // L
SLOT_CHUNKS = MAX_ACTIVE // L
CAP = 512
GRP = 8
NEG = np.float32(-3.0e38)


def _lanes():
  return lax.broadcasted_iota(jnp.int32, (L,), 0)


def _sc_body(pobj_hbm, pobjp_hbm, kbr_hbm, ksub_hbm, kobj_hbm, krel_hbm,
             sub_o, obj_o, rel_o, lesub_o, epsub_o, valid_o,
             src_v, rng_v, act_v, c_v, s0_v, nb_v,
             tsub_v, tobj_v, trel_v,
             sub_v, obj_v, rel_v, lesub_v, epsub_v, pobjp_v, valid_v,
             pobj_v, sem):
  wid = lax.axis_index("s") * 2 + lax.axis_index("c")

  @pl.when(wid < BSZ)
  def _():
    lane = _lanes()
    zero_chunk = jnp.zeros((L,), jnp.int32)
    pltpu.sync_copy(pobj_hbm.at[wid, 0], pobj_v)
    pltpu.sync_copy(pobjp_hbm.at[wid, 0], pobjp_v)

    zf = jnp.zeros((L,), jnp.float32)

    def zb(c, _):
      for u in range(5):
        src_v[pl.ds((c * 5 + u) * L, L)] = zf
      return 0

    lax.fori_loop(0, ENT_CHUNKS // 5, zb, 0)

    def sc_b(s, _):
      o = pobj_v[pl.ds(s * L, L)]
      p = pobjp_v[pl.ds(s * L, L)]
      plsc.addupdate_scatter(src_v, [o], p)
      return 0

    lax.fori_loop(0, SLOT_CHUNKS, sc_b, 0)

    head = src_v[pl.ds(0, L)]
    src_v[pl.ds(0, L)] = jnp.where(lane == 0, -1.0, head)

    def za(c, _):
      for u in range(8):
        act_v[pl.ds((c * 8 + u) * L, L)] = zero_chunk
      return 0

    lax.fori_loop(0, CAP // L // 8, za, 0)
    nb_v[pl.ds(0, L)] = zero_chunk

    def compact(c, _):
      v = src_v[pl.ds(c * L, L)]
      m = v > THRES

      @pl.when(jnp.any(m))
      def _():
        mi = m.astype(jnp.int32)
        na0 = nb_v[pl.ds(0, L)][0]
        cnt = plsc.all_reduce_population_count(m)
        pos = na0 + plsc.cumsum(mi) - 1
        plsc.store_scatter(act_v, [pos], c * L + lane, mask=m)
        nb_v[pl.ds(0, L)] = na0 + cnt

      return 0

    def group_body(g, _):
      base = g * GRP
      ms = [src_v[pl.ds((base + u) * L, L)] > THRES for u in range(GRP)]
      acc = ms[0]
      for u in range(1, GRP):
        acc = acc | ms[u]

      @pl.when(jnp.any(acc))
      def _():
        lax.fori_loop(base, base + GRP, compact, 0)

      return 0

    lax.fori_loop(0, ENT_CHUNKS // GRP, group_body, 0)
    lax.fori_loop(ENT_CHUNKS // GRP * GRP, ENT_CHUNKS, compact, 0)
    na = nb_v[pl.ds(0, L)][0]

    @pl.when(na == 0)
    def _():
      def fb(c, carry):
        best, bestidx = carry
        v = src_v[pl.ds(c * L, L)]
        cmax = jnp.max(v)
        carg = jnp.min(jnp.where(v == cmax, c * L + lane, np.int32(2 ** 30)))
        better = cmax > best
        return (jnp.where(better, cmax, best),
                jnp.where(better, carg, bestidx))

      _, bi = lax.fori_loop(0, ENT_CHUNKS, fb, (NEG, np.int32(1)))
      act_v[pl.ds(0, L)] = jnp.full((L,), bi, jnp.int32)

    k = jnp.maximum(na, 1)
    nk = lax.div(k + (L - 1), L)

    def kr_body(j, _):
      e = act_v[pl.ds(j, L)][0]
      pltpu.async_copy(kbr_hbm.at[pl.ds(e, 1)], rng_v.at[pl.ds(j, 1)], sem)
      return 0

    lax.fori_loop(0, k, kr_body, 0)

    def kr_drain(_, x):
      pltpu.make_async_copy(kbr_hbm.at[pl.ds(0, 1)],
                            rng_v.at[pl.ds(0, 1)], sem).wait()
      return x

    lax.fori_loop(0, k, kr_drain, 0)
    zi = jnp.full((L,), 0, jnp.int32)
    oi = jnp.full((L,), 1, jnp.int32)

    def rng_body(j, tot):
      gi = j * L + lane
      lm = gi < k
      r0 = plsc.load_gather(rng_v, [gi, zi])
      r1 = plsc.load_gather(rng_v, [gi, oi])
      ln = jnp.where(lm, r1 - r0, 0)
      inc = plsc.cumsum(ln)
      cex = tot + inc - ln
      cl = jnp.minimum(ln, jnp.maximum(MAX_ACTIVE - cex, 0))
      c_v[pl.ds(j * L, L)] = cex
      s0_v[pl.ds(j * L, L)] = r0
      nb_v[pl.ds(j * L, L)] = lax.div(cl + (L - 1), L)
      return tot + jnp.max(inc)

    total = lax.fori_loop(0, nk, rng_body, np.int32(0))
    count = jnp.minimum(total, MAX_ACTIVE)

    def ent_body(j, nd):
      nb = nb_v[pl.ds(j, L)][0]
      c0 = c_v[pl.ds(j, L)][0]
      s0 = s0_v[pl.ds(j, L)][0]

      def blk(q, nd2):
        pltpu.async_copy(ksub_hbm.at[pl.ds(s0 + q * L, L)],
                         tsub_v.at[pl.ds(c0 + q * L, L)], sem)
        pltpu.async_copy(kobj_hbm.at[pl.ds(s0 + q * L, L)],
                         tobj_v.at[pl.ds(c0 + q * L, L)], sem)
        pltpu.async_copy(krel_hbm.at[pl.ds(s0 + q * L, L)],
                         trel_v.at[pl.ds(c0 + q * L, L)], sem)
        return nd2 + 1

      return lax.fori_loop(0, nb, blk, nd)

    ndma = lax.fori_loop(0, k, ent_body, np.int32(0))

    def drain(_, x):
      pltpu.make_async_copy(ksub_hbm.at[pl.ds(0, L)],
                            tsub_v.at[pl.ds(0, L)], sem).wait()
      pltpu.make_async_copy(kobj_hbm.at[pl.ds(0, L)],
                            tobj_v.at[pl.ds(0, L)], sem).wait()
      pltpu.make_async_copy(krel_hbm.at[pl.ds(0, L)],
                            trel_v.at[pl.ds(0, L)], sem).wait()
      return x

    lax.fori_loop(0, ndma, drain, 0)

    def col_body(s, _):
      p = s * L + lane
      sub = jnp.clip(plsc.load_gather(tsub_v, [p, zi]), 0, NUM_ENT - 1)
      obj = jnp.clip(plsc.load_gather(tobj_v, [p, zi]), 0, NUM_ENT - 1)
      rel = jnp.clip(plsc.load_gather(trel_v, [p, zi]), 0, NUM_REL - 1)
      sub_v[pl.ds(s * L, L)] = sub
      obj_v[pl.ds(s * L, L)] = obj
      rel_v[pl.ds(s * L, L)] = rel
      sv = plsc.load_gather(src_v, [sub])
      epsub_v[pl.ds(s * L, L)] = sv
      lesub_v[pl.ds(s * L, L)] = jnp.minimum(sv, 1.0)
      valid_v[pl.ds(s * L, L)] = (p < count).astype(jnp.float32)
      return 0

    lax.fori_loop(0, SLOT_CHUNKS, col_body, 0)

    for vec, hbm in [(sub_v, sub_o), (obj_v, obj_o), (rel_v, rel_o),
                     (lesub_v, lesub_o), (epsub_v, epsub_o),
                     (valid_v, valid_o)]:
      pltpu.sync_copy(vec, hbm.at[wid, 0])


@functools.lru_cache(maxsize=1)
def _build_sc_kernel():
  mesh = plsc.VectorSubcoreMesh(core_axis_name="c", subcore_axis_name="s")
  cp = pltpu.CompilerParams(needs_layout_passes=False,
                            use_tc_tiling_on_sc=False)
  rows_i = jax.ShapeDtypeStruct((BSZ, 1, MAX_ACTIVE), jnp.int32)
  rows_f = jax.ShapeDtypeStruct((BSZ, 1, MAX_ACTIVE), jnp.float32)
  return pl.kernel(
      _sc_body, mesh=mesh,
      out_type=(rows_i, rows_i, rows_i, rows_f, rows_f, rows_f),
      scratch_types=[
          pltpu.VMEM((NUM_ENT,), jnp.float32),
          pltpu.VMEM((CAP, 2), jnp.int32),
          pltpu.VMEM((CAP,), jnp.int32),
          pltpu.VMEM((CAP,), jnp.int32),
          pltpu.VMEM((CAP,), jnp.int32),
          pltpu.VMEM((CAP,), jnp.int32),
          pltpu.VMEM((CAP, 1), jnp.int32),
          pltpu.VMEM((CAP, 1), jnp.int32),
          pltpu.VMEM((CAP, 1), jnp.int32),
          pltpu.VMEM((MAX_ACTIVE,), jnp.int32),
          pltpu.VMEM((MAX_ACTIVE,), jnp.int32),
          pltpu.VMEM((MAX_ACTIVE,), jnp.int32),
          pltpu.VMEM((MAX_ACTIVE,), jnp.float32),
          pltpu.VMEM((MAX_ACTIVE,), jnp.float32),
          pltpu.VMEM((MAX_ACTIVE,), jnp.float32),
          pltpu.VMEM((MAX_ACTIVE,), jnp.float32),
          pltpu.VMEM((MAX_ACTIVE,), jnp.int32),
          pltpu.SemaphoreType.DMA,
      ], compiler_params=cp)


def _gates(gi, gh):
  i_r, i_z, i_n = gi[:, :DIM], gi[:, DIM:2 * DIM], gi[:, 2 * DIM:]
  h_r, h_z, h_n = gh[:, :DIM], gh[:, DIM:2 * DIM], gh[:, 2 * DIM:]
  r = jax.nn.sigmoid(i_r + h_r)
  z = jax.nn.sigmoid(i_z + h_z)
  n = jnp.tanh(i_n + r * h_n)
  return z, n


def _tc1_body(rel_ref, lesub_ref, valid_ref, q1h_ref,
              rel_emb_ref, W_ih_ref, b_ih_ref, b_hh_ref,
              sw_ref, sb_ref, clsw_ref, clsb_ref,
              objp_ref, feat_ref):
  rel = rel_ref[0, 0]
  oh = (rel[:, None] ==
        lax.broadcasted_iota(jnp.int32, (MAX_ACTIVE, NUM_REL), 1)
        ).astype(jnp.float32)
  rel_feat = jnp.dot(oh, rel_emb_ref[...],
                     preferred_element_type=jnp.float32)
  gi = jnp.dot(rel_feat, W_ih_ref[...],
               preferred_element_type=jnp.float32) + b_ih_ref[...]
  gh = jnp.broadcast_to(b_hh_ref[...], (MAX_ACTIVE, 3 * DIM))
  z, n = _gates(gi, gh)
  trans = (1.0 - z) * n
  qe = jnp.dot(q1h_ref[0], rel_emb_ref[...],
               preferred_element_type=jnp.float32)
  cq = jnp.tanh(jnp.dot(qe, sw_ref[...],
                        preferred_element_type=jnp.float32) + sb_ref[...])
  logit = jnp.sum(trans * cq * clsw_ref[...], axis=1) + clsb_ref[0, 0]
  prob = jax.nn.sigmoid(logit)
  obj_p = lesub_ref[0, 0] * prob * valid_ref[0, 0]
  objp_ref[0, 0] = obj_p
  feat_ref[0] = trans * obj_p[:, None]


def _tc2_body(rel_ref, sub_ref, lesub_ref, epsub_ref, valid_ref,
              pobj_ref, pfeat_ref, obj_ref, q1h_ref,
              rel_emb_ref, W_ih_ref, W_hh_ref, b_ih_ref, b_hh_ref,
              sw_ref, sb_ref, clsw_ref, clsb_ref,
              out_ref):
  rel = rel_ref[0, 0]
  oh = (rel[:, None] ==
        lax.broadcasted_iota(jnp.int32, (MAX_ACTIVE, NUM_REL), 1)
        ).astype(jnp.float32)
  rel_feat = jnp.dot(oh, rel_emb_ref[...],
                     preferred_element_type=jnp.float32)
  gi = jnp.dot(rel_feat, W_ih_ref[...],
               preferred_element_type=jnp.float32) + b_ih_ref[...]
  mask = (sub_ref[0, 0][:, None] == pobj_ref[0, 0][None, :]
          ).astype(jnp.float32)
  S = jnp.dot(mask, pfeat_ref[0], preferred_element_type=jnp.float32)
  h = S / (epsub_ref[0, 0][:, None] + 1e-6)
  gh = jnp.dot(h, W_hh_ref[...],
               preferred_element_type=jnp.float32) + b_hh_ref[...]
  z, n = _gates(gi, gh)
  trans = (1.0 - z) * n + z * h
  qe = jnp.dot(q1h_ref[0], rel_emb_ref[...],
               preferred_element_type=jnp.float32)
  cq = jnp.tanh(jnp.dot(qe, sw_ref[...],
                        preferred_element_type=jnp.float32) + sb_ref[...])
  logit = jnp.sum(trans * cq * clsw_ref[...], axis=1) + clsb_ref[0, 0]
  prob = jax.nn.sigmoid(logit)
  obj_p = lesub_ref[0, 0] * prob * valid_ref[0, 0]
  obj = obj_ref[0, 0]
  hi = obj_p.astype(jnp.bfloat16).astype(jnp.float32)
  lo = obj_p - hi
  hl = jnp.stack([hi, lo], axis=0)
  CH = 2000
  for c in range(NUM_ENT // CH):
    ids = c * CH + lax.broadcasted_iota(jnp.int32, (MAX_ACTIVE, CH), 1)
    ohc = (obj[:, None] == ids).astype(jnp.float32)
    contrib = jnp.dot(hl, ohc, preferred_element_type=jnp.float32)
    out_ref[0, 0, c * CH:(c + 1) * CH] = jnp.minimum(contrib[0] + contrib[1],
                                                     1.0)


_B3 = lambda: pl.BlockSpec((1, 1, MAX_ACTIVE), lambda i: (i, 0, 0))
_BQ = lambda: pl.BlockSpec((1, 1, NUM_REL), lambda i: (i, 0, 0))
_BW = lambda shape: pl.BlockSpec(shape, lambda i: tuple(0 for _ in shape))


def kernel(start, query, kb_triple, kb_range, rel_emb, step_W, step_b,
           cls_W, cls_b, W_ih, W_hh, b_ih, b_hh):
  f32 = jnp.float32
  i32 = jnp.int32
  kb32 = kb_triple.astype(i32)
  ksub = jnp.pad(kb32[:, 0], (0, L)).reshape(N_TRIPLES + L, 1)
  kobj = jnp.pad(kb32[:, 1], (0, L)).reshape(N_TRIPLES + L, 1)
  krel = jnp.pad(kb32[:, 2], (0, L)).reshape(N_TRIPLES + L, 1)
  kbr = kb_range.astype(i32)
  q1h = (query[:, None].astype(i32) ==
         jnp.arange(NUM_REL, dtype=i32)[None, :]).astype(f32)
  q1h = q1h.reshape(BSZ, 1, NUM_REL)
  b_ih2 = b_ih.reshape(1, 3 * DIM).astype(f32)
  b_hh2 = b_hh.reshape(1, 3 * DIM).astype(f32)
  clsw = cls_W.reshape(1, DIM).astype(f32)
  clsb = cls_b.reshape(1, 1).astype(f32)
  sb = step_b.reshape(2, 1, DIM).astype(f32)
  W_ihT = W_ih.astype(f32).T
  W_hhT = W_hh.astype(f32).T
  sc_sel = _build_sc_kernel()

  e0 = jnp.argmax(start, axis=1).astype(i32)
  v0 = jnp.max(start.astype(f32), axis=1)
  obj0 = jnp.broadcast_to(e0[:, None], (BSZ, MAX_ACTIVE)).reshape(
      BSZ, 1, MAX_ACTIVE)
  objp0 = jnp.pad(v0[:, None], ((0, 0), (0, MAX_ACTIVE - 1))).reshape(
      BSZ, 1, MAX_ACTIVE)

  _, obj1, rel1, lesub1, _, valid1 = sc_sel(obj0, objp0, kbr, ksub, kobj, krel)

  tc1 = pl.pallas_call(
      _tc1_body,
      grid=(BSZ,),
      in_specs=[_B3(), _B3(), _B3(), _BQ(),
                _BW((NUM_REL, DIM)), _BW((DIM, 3 * DIM)), _BW((1, 3 * DIM)),
                _BW((1, 3 * DIM)), _BW((DIM, DIM)), _BW((1, DIM)),
                _BW((1, DIM)), _BW((1, 1))],
      out_specs=[_B3(),
                 pl.BlockSpec((1, MAX_ACTIVE, DIM), lambda i: (i, 0, 0))],
      out_shape=[jax.ShapeDtypeStruct((BSZ, 1, MAX_ACTIVE), f32),
                 jax.ShapeDtypeStruct((BSZ, MAX_ACTIVE, DIM), f32)],
  )
  objp1, feat1 = tc1(rel1, lesub1, valid1, q1h,
                     rel_emb.astype(f32), W_ihT, b_ih2, b_hh2,
                     step_W[0].astype(f32), sb[0], clsw, clsb)

  sub2, obj2, rel2, lesub2, epsub2, valid2 = sc_sel(obj1, objp1, kbr, ksub, kobj, krel)

  tc2 = pl.pallas_call(
      _tc2_body,
      grid=(BSZ,),
      in_specs=[_B3(), _B3(), _B3(), _B3(), _B3(), _B3(),
                pl.BlockSpec((1, MAX_ACTIVE, DIM), lambda i: (i, 0, 0)),
                _B3(), _BQ(),
                _BW((NUM_REL, DIM)), _BW((DIM, 3 * DIM)), _BW((DIM, 3 * DIM)),
                _BW((1, 3 * DIM)), _BW((1, 3 * DIM)),
                _BW((DIM, DIM)), _BW((1, DIM)), _BW((1, DIM)), _BW((1, 1))],
      out_specs=[pl.BlockSpec((1, 1, NUM_ENT), lambda i: (i, 0, 0))],
      out_shape=[jax.ShapeDtypeStruct((BSZ, 1, NUM_ENT), f32)],
  )
  (out,) = tc2(rel2, sub2, lesub2, epsub2, valid2, obj1, feat1, obj2, q1h,
               rel_emb.astype(f32), W_ihT, W_hhT, b_ih2, b_hh2,
               step_W[1].astype(f32), sb[1], clsw, clsb)
  return out.reshape(BSZ, NUM_ENT)


# speedup vs baseline: 2.7595x; 2.7595x over previous
import functools
import jax
import jax.numpy as jnp
import numpy as np
from jax import lax
from jax.experimental import pallas as pl
from jax.experimental.pallas import tpu as pltpu
from jax.experimental.pallas import tpu_sc as plsc

DIM = 128
NUM_ENT = 10000
NUM_REL = 200
N_TRIPLES = 160000
BSZ = 16
MAX_ACTIVE = 400
THRES = 0.7
L = 16
ENT_CHUNKS = NUM_ENT // L
SLOT_CHUNKS = MAX_ACTIVE // L
CAP = 512
NEG = np.float32(-3.0e38)

def _lanes():
  return lax.broadcasted_iota(jnp.int32, (L,), 0)


def _select_and_gather(src_v, r0_v, r1_v, kbf_hbm, act_v, c_v, s0_v, nb_v,
                       tri_v, sem,
                       sub_v, obj_v, rel_v, lesub_v, epsub_v, valid_v,
                       with_ep):
  lane = _lanes()
  zero_chunk = jnp.zeros((L,), jnp.int32)

  def zero_act(c, _):
    for u in range(8):
      act_v[pl.ds((c * 8 + u) * L, L)] = zero_chunk
    return 0

  lax.fori_loop(0, CAP // L // 8, zero_act, 0)

  head = src_v[pl.ds(0, L)]
  src_v[pl.ds(0, L)] = jnp.where(lane == 0, -1.0, head)

  nb_v[pl.ds(0, L)] = zero_chunk

  GRP = 8
  NGRP = ENT_CHUNKS // GRP

  def compact_chunk(c, _):
    v = src_v[pl.ds(c * L, L)]
    m = v > THRES

    @pl.when(jnp.any(m))
    def _():
      mi = m.astype(jnp.int32)
      na0 = nb_v[pl.ds(0, L)][0]
      cnt = plsc.all_reduce_population_count(m)
      pos = na0 + plsc.cumsum(mi) - 1
      plsc.store_scatter(act_v, [pos], c * L + lane, mask=m)
      nb_v[pl.ds(0, L)] = na0 + cnt

    return 0

  def group_body(g, _):
    base = g * GRP
    ms = [src_v[pl.ds((base + u) * L, L)] > THRES for u in range(GRP)]
    acc = ms[0]
    for u in range(1, GRP):
      acc = acc | ms[u]

    @pl.when(jnp.any(acc))
    def _():
      lax.fori_loop(base, base + GRP, compact_chunk, 0)

    return 0

  lax.fori_loop(0, NGRP, group_body, 0)
  lax.fori_loop(NGRP * GRP, ENT_CHUNKS, compact_chunk, 0)
  na = nb_v[pl.ds(0, L)][0]

  @pl.when(na == 0)
  def _():
    def fb_body(c, carry):
      best, bestidx = carry
      v = src_v[pl.ds(c * L, L)]
      cmax = jnp.max(v)
      carg = jnp.min(jnp.where(v == cmax, c * L + lane, np.int32(2 ** 30)))
      better = cmax > best
      return (jnp.where(better, cmax, best),
              jnp.where(better, carg, bestidx))

    _, bestidx = lax.fori_loop(0, ENT_CHUNKS, fb_body, (NEG, np.int32(1)))
    act_v[pl.ds(0, L)] = jnp.full((L,), bestidx, jnp.int32)

  k = jnp.maximum(na, 1)
  nk = lax.div(k + (L - 1), L)

  def rng_body(j, tot):
    gi = j * L + lane
    lm = gi < k
    e = act_v[pl.ds(j * L, L)]
    r0 = plsc.load_gather(r0_v, [e])
    r1 = plsc.load_gather(r1_v, [e])
    ln = jnp.where(lm, r1 - r0, 0)
    inc = plsc.cumsum(ln)
    cex = tot + inc - ln
    room = jnp.maximum(MAX_ACTIVE - cex, 0)
    cl = jnp.minimum(ln, room)
    c_v[pl.ds(j * L, L)] = cex
    s0_v[pl.ds(j * L, L)] = r0
    nb_v[pl.ds(j * L, L)] = lax.div(cl + (L - 1), L)
    return tot + jnp.max(inc)

  total = lax.fori_loop(0, nk, rng_body, np.int32(0))
  count = jnp.minimum(total, MAX_ACTIVE)

  def ent_body(j, nd):
    nb = nb_v[pl.ds(j, L)][0]
    c0 = c_v[pl.ds(j, L)][0]
    s0 = s0_v[pl.ds(j, L)][0]

    def blk(q, nd2):
      pltpu.async_copy(kbf_hbm.at[pl.ds((s0 + q * L) * 8, 128)],
                       tri_v.at[pl.ds((c0 + q * L) * 8, 128)], sem)
      return nd2 + 1

    return lax.fori_loop(0, nb, blk, nd)

  ndma = lax.fori_loop(0, k, ent_body, np.int32(0))

  def drain(_, x):
    pltpu.make_async_copy(kbf_hbm.at[pl.ds(0, 128)],
                          tri_v.at[pl.ds(0, 128)], sem).wait()
    return x

  lax.fori_loop(0, ndma, drain, 0)

  def col_body(s, _):
    p = (s * L + lane) * 8
    sub = jnp.clip(plsc.load_gather(tri_v, [p]), 0, NUM_ENT - 1)
    obj = jnp.clip(plsc.load_gather(tri_v, [p + 1]), 0, NUM_ENT - 1)
    rel = jnp.clip(plsc.load_gather(tri_v, [p + 2]), 0, NUM_REL - 1)
    sub_v[pl.ds(s * L, L)] = sub
    obj_v[pl.ds(s * L, L)] = obj
    rel_v[pl.ds(s * L, L)] = rel
    sv = plsc.load_gather(src_v, [sub])
    if with_ep:
      epsub_v[pl.ds(s * L, L)] = sv
      lesub_v[pl.ds(s * L, L)] = jnp.minimum(sv, 1.0)
    else:
      lesub_v[pl.ds(s * L, L)] = sv
    valid_v[pl.ds(s * L, L)] = ((s * L + lane) < count).astype(jnp.float32)
    return 0

  lax.fori_loop(0, SLOT_CHUNKS, col_body, 0)


def _scatter_ep(ep_v, objrow_v, objprow_v):
  zf = jnp.zeros((L,), jnp.float32)

  def zb(c, _):
    for u in range(5):
      ep_v[pl.ds((c * 5 + u) * L, L)] = zf
    return 0

  lax.fori_loop(0, ENT_CHUNKS // 5, zb, 0)

  def sc_body(s, _):
    o = objrow_v[pl.ds(s * L, L)]
    p = objprow_v[pl.ds(s * L, L)]
    plsc.addupdate_scatter(ep_v, [o], p)
    return 0

  lax.fori_loop(0, SLOT_CHUNKS, sc_body, 0)


@functools.lru_cache(maxsize=1)
def _build_sc_kernels():
  mesh = plsc.VectorSubcoreMesh(core_axis_name="c", subcore_axis_name="s")
  cp = pltpu.CompilerParams(needs_layout_passes=False)
  sel_scratch = [
      pltpu.VMEM((NUM_ENT,), jnp.float32),
      pltpu.VMEM((NUM_ENT,), jnp.int32),
      pltpu.VMEM((NUM_ENT,), jnp.int32),
      pltpu.VMEM((CAP,), jnp.int32),
      pltpu.VMEM((CAP,), jnp.int32),
      pltpu.VMEM((CAP,), jnp.int32),
      pltpu.VMEM((CAP,), jnp.int32),
      pltpu.VMEM((CAP * 8,), jnp.int32),
      pltpu.VMEM((MAX_ACTIVE,), jnp.int32),
      pltpu.VMEM((MAX_ACTIVE,), jnp.int32),
      pltpu.VMEM((MAX_ACTIVE,), jnp.int32),
      pltpu.VMEM((MAX_ACTIVE,), jnp.float32),
      pltpu.VMEM((MAX_ACTIVE,), jnp.float32),
      pltpu.VMEM((MAX_ACTIVE,), jnp.float32),
      pltpu.VMEM((MAX_ACTIVE,), jnp.float32),
      pltpu.SemaphoreType.DMA,
  ]
  rows_i = jax.ShapeDtypeStruct((BSZ, MAX_ACTIVE), jnp.int32)
  rows_f = jax.ShapeDtypeStruct((BSZ, MAX_ACTIVE), jnp.float32)

  def out_rows(i, pairs):
    for vec, hbm in pairs:
      pltpu.sync_copy(vec, hbm.at[i])

  @functools.partial(
      pl.kernel, mesh=mesh,
      out_type=(rows_i, rows_i, rows_i, rows_f, rows_f),
      scratch_types=sel_scratch, compiler_params=cp)
  def sc1(start_hbm, r0_hbm, r1_hbm, kbf_hbm,
          sub_o, obj_o, rel_o, lesub_o, valid_o,
          src_v, r0_v, r1_v, act_v, c_v, s0_v, nb_v, tri_v,
          sub_v, obj_v, rel_v, lesub_v, epsub_v, objprow_v, valid_v, sem):
    wid = lax.axis_index("s") * 2 + lax.axis_index("c")

    @pl.when(wid < BSZ)
    def _():
      cp0 = pltpu.async_copy(r0_hbm, r0_v, sem)
      cp1 = pltpu.async_copy(r1_hbm, r1_v, sem)
      pltpu.sync_copy(start_hbm.at[wid], src_v)
      cp0.wait()
      cp1.wait()
      _select_and_gather(src_v, r0_v, r1_v, kbf_hbm, act_v, c_v, s0_v, nb_v,
                         tri_v, sem,
                         sub_v, obj_v, rel_v, lesub_v, epsub_v, valid_v,
                         with_ep=False)
      out_rows(wid, [(sub_v, sub_o), (obj_v, obj_o), (rel_v, rel_o),
                     (lesub_v, lesub_o), (valid_v, valid_o)])

  @functools.partial(
      pl.kernel, mesh=mesh,
      out_type=(rows_i, rows_i, rows_i, rows_f, rows_f, rows_f),
      scratch_types=sel_scratch + [pltpu.VMEM((MAX_ACTIVE,), jnp.int32)],
      compiler_params=cp)
  def sc2(obj_hbm, objp_hbm, r0_hbm, r1_hbm, kbf_hbm,
          sub_o, obj_o, rel_o, lesub_o, epsub_o, valid_o,
          src_v, r0_v, r1_v, act_v, c_v, s0_v, nb_v, tri_v,
          sub_v, obj_v, rel_v, lesub_v, epsub_v, objprow_v, valid_v, sem,
          pobj_v):
    wid = lax.axis_index("s") * 2 + lax.axis_index("c")

    @pl.when(wid < BSZ)
    def _():
      cp0 = pltpu.async_copy(r0_hbm, r0_v, sem)
      cp1 = pltpu.async_copy(r1_hbm, r1_v, sem)
      pltpu.sync_copy(obj_hbm.at[wid], pobj_v)
      pltpu.sync_copy(objp_hbm.at[wid], objprow_v)
      _scatter_ep(src_v, pobj_v, objprow_v)
      cp0.wait()
      cp1.wait()
      _select_and_gather(src_v, r0_v, r1_v, kbf_hbm, act_v, c_v, s0_v, nb_v,
                         tri_v, sem,
                         sub_v, obj_v, rel_v, lesub_v, epsub_v, valid_v,
                         with_ep=True)
      out_rows(wid, [(sub_v, sub_o), (obj_v, obj_o), (rel_v, rel_o),
                     (lesub_v, lesub_o), (epsub_v, epsub_o),
                     (valid_v, valid_o)])

  @functools.partial(
      pl.kernel, mesh=mesh,
      out_type=jax.ShapeDtypeStruct((BSZ, NUM_ENT), jnp.float32),
      scratch_types=[
          pltpu.VMEM((NUM_ENT,), jnp.float32),
          pltpu.VMEM((MAX_ACTIVE,), jnp.int32),
          pltpu.VMEM((MAX_ACTIVE,), jnp.float32),
      ], compiler_params=cp)
  def sc3(obj_hbm, objp_hbm, out_hbm, ep_v, objrow_v, objprow_v):
    wid = lax.axis_index("s") * 2 + lax.axis_index("c")

    @pl.when(wid < BSZ)
    def _():
      pltpu.sync_copy(obj_hbm.at[wid], objrow_v)
      pltpu.sync_copy(objp_hbm.at[wid], objprow_v)
      _scatter_ep(ep_v, objrow_v, objprow_v)

      def norm_body(c, _):
        for u in range(5):
          o = (c * 5 + u) * L
          ep_v[pl.ds(o, L)] = jnp.minimum(ep_v[pl.ds(o, L)], 1.0)
        return 0

      lax.fori_loop(0, ENT_CHUNKS // 5, norm_body, 0)
      pltpu.sync_copy(ep_v, out_hbm.at[wid])

  return sc1, sc2, sc3


def _gates(gi, gh):
  i_r, i_z, i_n = gi[:, :DIM], gi[:, DIM:2 * DIM], gi[:, 2 * DIM:]
  h_r, h_z, h_n = gh[:, :DIM], gh[:, DIM:2 * DIM], gh[:, 2 * DIM:]
  r = jax.nn.sigmoid(i_r + h_r)
  z = jax.nn.sigmoid(i_z + h_z)
  n = jnp.tanh(i_n + r * h_n)
  return z, n


def _tc1_body(rel_ref, lesub_ref, valid_ref, q1h_ref,
              rel_emb_ref, W_ih_ref, b_ih_ref, b_hh_ref,
              sw_ref, sb_ref, clsw_ref, clsb_ref,
              objp_ref, feat_ref):
  rel = rel_ref[0, 0]
  oh = (rel[:, None] ==
        lax.broadcasted_iota(jnp.int32, (MAX_ACTIVE, NUM_REL), 1)
        ).astype(jnp.float32)
  rel_feat = jnp.dot(oh, rel_emb_ref[...],
                     preferred_element_type=jnp.float32)
  gi = jnp.dot(rel_feat, W_ih_ref[...],
               preferred_element_type=jnp.float32) + b_ih_ref[...]
  gh = jnp.broadcast_to(b_hh_ref[...], (MAX_ACTIVE, 3 * DIM))
  z, n = _gates(gi, gh)
  trans = (1.0 - z) * n
  qe = jnp.dot(q1h_ref[0], rel_emb_ref[...],
               preferred_element_type=jnp.float32)
  cq = jnp.tanh(jnp.dot(qe, sw_ref[...],
                        preferred_element_type=jnp.float32) + sb_ref[...])
  logit = jnp.sum(trans * cq * clsw_ref[...], axis=1) + clsb_ref[0, 0]
  prob = jax.nn.sigmoid(logit)
  obj_p = lesub_ref[0, 0] * prob * valid_ref[0, 0]
  objp_ref[0, 0] = obj_p
  feat_ref[0] = trans * obj_p[:, None]


def _tc2_body(rel_ref, sub_ref, lesub_ref, epsub_ref, valid_ref,
              pobj_ref, pfeat_ref, q1h_ref,
              rel_emb_ref, W_ih_ref, W_hh_ref, b_ih_ref, b_hh_ref,
              sw_ref, sb_ref, clsw_ref, clsb_ref,
              objp_ref):
  rel = rel_ref[0, 0]
  oh = (rel[:, None] ==
        lax.broadcasted_iota(jnp.int32, (MAX_ACTIVE, NUM_REL), 1)
        ).astype(jnp.float32)
  rel_feat = jnp.dot(oh, rel_emb_ref[...],
                     preferred_element_type=jnp.float32)
  gi = jnp.dot(rel_feat, W_ih_ref[...],
               preferred_element_type=jnp.float32) + b_ih_ref[...]
  mask = (sub_ref[0, 0][:, None] == pobj_ref[0, 0][None, :]
          ).astype(jnp.float32)
  S = jnp.dot(mask, pfeat_ref[0], preferred_element_type=jnp.float32)
  h = S / (epsub_ref[0, 0][:, None] + 1e-6)
  gh = jnp.dot(h, W_hh_ref[...],
               preferred_element_type=jnp.float32) + b_hh_ref[...]
  z, n = _gates(gi, gh)
  trans = (1.0 - z) * n + z * h
  qe = jnp.dot(q1h_ref[0], rel_emb_ref[...],
               preferred_element_type=jnp.float32)
  cq = jnp.tanh(jnp.dot(qe, sw_ref[...],
                        preferred_element_type=jnp.float32) + sb_ref[...])
  logit = jnp.sum(trans * cq * clsw_ref[...], axis=1) + clsb_ref[0, 0]
  prob = jax.nn.sigmoid(logit)
  objp_ref[0, 0] = lesub_ref[0, 0] * prob * valid_ref[0, 0]


def _i3(x):
  return x.reshape(BSZ, 1, MAX_ACTIVE)


_B3 = lambda: pl.BlockSpec((1, 1, MAX_ACTIVE), lambda i: (i, 0, 0))
_BQ = lambda: pl.BlockSpec((1, 1, NUM_REL), lambda i: (i, 0, 0))
_BW = lambda shape: pl.BlockSpec(shape, lambda i: tuple(0 for _ in shape))


def kernel(start, query, kb_triple, kb_range, rel_emb, step_W, step_b,
           cls_W, cls_b, W_ih, W_hh, b_ih, b_hh):
  f32 = jnp.float32
  kbf = jnp.pad(kb_triple.astype(jnp.int32), ((0, 16), (0, 5))).reshape(-1)
  r0 = kb_range[:, 0].astype(jnp.int32)
  r1 = kb_range[:, 1].astype(jnp.int32)
  q1h = (query[:, None].astype(jnp.int32) ==
         jnp.arange(NUM_REL, dtype=jnp.int32)[None, :]).astype(f32)
  q1h = q1h.reshape(BSZ, 1, NUM_REL)
  b_ih2 = b_ih.reshape(1, 3 * DIM).astype(f32)
  b_hh2 = b_hh.reshape(1, 3 * DIM).astype(f32)
  clsw = cls_W.reshape(1, DIM).astype(f32)
  clsb = cls_b.reshape(1, 1).astype(f32)
  sb = step_b.reshape(2, 1, DIM).astype(f32)
  _sc1, _sc2, _sc3 = _build_sc_kernels()

  sub1, obj1, rel1, lesub1, valid1 = _sc1(start.astype(f32), r0, r1, kbf)

  tc1 = pl.pallas_call(
      _tc1_body,
      grid=(BSZ,),
      in_specs=[_B3(), _B3(), _B3(), _BQ(),
                _BW((NUM_REL, DIM)), _BW((DIM, 3 * DIM)), _BW((1, 3 * DIM)),
                _BW((1, 3 * DIM)), _BW((DIM, DIM)), _BW((1, DIM)),
                _BW((1, DIM)), _BW((1, 1))],
      out_specs=[_B3(), pl.BlockSpec((1, MAX_ACTIVE, DIM), lambda i: (i, 0, 0))],
      out_shape=[jax.ShapeDtypeStruct((BSZ, 1, MAX_ACTIVE), f32),
                 jax.ShapeDtypeStruct((BSZ, MAX_ACTIVE, DIM), f32)],
  )
  objp1, feat1 = tc1(_i3(rel1), _i3(lesub1), _i3(valid1), q1h,
                     rel_emb.astype(f32), W_ih.astype(f32).T, b_ih2, b_hh2,
                     step_W[0].astype(f32), sb[0], clsw, clsb)

  sub2, obj2, rel2, lesub2, epsub2, valid2 = _sc2(
      obj1, objp1.reshape(BSZ, MAX_ACTIVE), r0, r1, kbf)

  tc2 = pl.pallas_call(
      _tc2_body,
      grid=(BSZ,),
      in_specs=[_B3(), _B3(), _B3(), _B3(), _B3(), _B3(),
                pl.BlockSpec((1, MAX_ACTIVE, DIM), lambda i: (i, 0, 0)), _BQ(),
                _BW((NUM_REL, DIM)), _BW((DIM, 3 * DIM)), _BW((DIM, 3 * DIM)),
                _BW((1, 3 * DIM)), _BW((1, 3 * DIM)),
                _BW((DIM, DIM)), _BW((1, DIM)), _BW((1, DIM)), _BW((1, 1))],
      out_specs=[_B3()],
      out_shape=[jax.ShapeDtypeStruct((BSZ, 1, MAX_ACTIVE), f32)],
  )
  (objp2,) = tc2(_i3(rel2), _i3(sub2), _i3(lesub2), _i3(epsub2), _i3(valid2),
                 _i3(obj1), feat1, q1h,
                 rel_emb.astype(f32), W_ih.astype(f32).T, W_hh.astype(f32).T,
                 b_ih2, b_hh2, step_W[1].astype(f32), sb[1], clsw, clsb)

  return _sc3(obj2, objp2.reshape(BSZ, MAX_ACTIVE))
